# baseline (speedup 1.0000x reference)
import functools

import jax
import jax.numpy as jnp
from jax import lax
from jax.experimental import pallas as pl
from jax.experimental.pallas import tpu as pltpu
from jax.experimental.pallas import tpu_sc as plsc

N = 10000
E = 320000
IN = 16
OUT = 16
D_EDGE = 16
HID = 64

NC = 2
NS = 16
NW = NC * NS
LANE = 128
RPW = 80
E_PAD = NW * RPW * LANE
EW = E_PAD // 8
PAD = E_PAD - E

G_CH = 4
GA = 112
GB = 48
GIDX = NS * GA + NS * GB + (GA - GB)
S_CH = 8

N_ACC = 10240
NAW = N_ACC * 16 // 128
STRIPE = N_ACC // NS
SW = STRIPE * 16 // 128

BE = 4096
BEW = BE // 8
NB = (E + BE - 1) // BE

_f32 = jnp.float32


def _make_gather(ga, gb, ne, row0):
  def _gather_body(x_hbm, srcidx_hbm, xj_hbm, idx_v, gbuf0, gbuf1, gsem,
                   osem0, osem1):
    c = lax.axis_index("c")
    s = lax.axis_index("s")
    local_row = jnp.where(c == 0, s * ga, NS * ga + s * gb)
    n_outer = jnp.where(c == 0, ga // (2 * G_CH), gb // (2 * G_CH))
    pltpu.sync_copy(srcidx_hbm.at[pl.ds(row0 + local_row, ga)], idx_v)
    gbufs = (gbuf0, gbuf1)
    osems = (osem0, osem1)

    def outer(k2, carry):
        for b in range(2):
            kk = k2 * 2 + b
            gb = gbufs[b]
            os_ = osems[b]

            @pl.when(kk >= 2)
            def _drain():
                pltpu.make_async_copy(
                    gb, xj_hbm.at[pl.ds(0, G_CH * LANE), pl.ds(0, IN)],
                    os_).wait()

            descs = []
            for r in range(G_CH):
                descs.append(
                    pltpu.async_copy(
                        x_hbm.at[idx_v.at[kk * G_CH + r]],
                        gb.at[pl.ds(r * LANE, LANE)],
                        gsem,
                    )
                )
            for d in descs:
                d.wait()
            pltpu.async_copy(
                gb,
                xj_hbm.at[pl.ds((local_row + kk * G_CH) * LANE, G_CH * LANE),
                          pl.ds(0, IN)],
                os_,
            )
        return carry

    lax.fori_loop(0, n_outer, outer, 0)
    for b in range(2):
        pltpu.make_async_copy(
            gbufs[b], xj_hbm.at[pl.ds(0, G_CH * LANE), pl.ds(0, IN)],
            osems[b]).wait()

  return functools.partial(
      pl.kernel,
      out_type=jax.ShapeDtypeStruct((ne * LANE, 128), _f32),
      mesh=plsc.VectorSubcoreMesh(core_axis_name="c", subcore_axis_name="s"),
      scratch_types=[
          pltpu.VMEM((ga, LANE), jnp.int32),
          pltpu.VMEM((G_CH * LANE, IN), _f32),
          pltpu.VMEM((G_CH * LANE, IN), _f32),
          pltpu.SemaphoreType.DMA,
          pltpu.SemaphoreType.DMA,
          pltpu.SemaphoreType.DMA,
      ],
      compiler_params=pltpu.CompilerParams(use_tc_tiling_on_sc=False),
  )(_gather_body)


def _msgs_body(ea, xj, w1, b1, w2, b2, rmat, smat, out):
    h = jnp.maximum(
        jnp.dot(ea[...], w1[...], preferred_element_type=_f32) + b1[...], 0.0
    )
    wflat = jnp.dot(h, w2[...], preferred_element_type=_f32) + b2[...]
    xt = jnp.dot(xj[:, 0:IN], rmat[...], preferred_element_type=_f32)
    out[:, 0:OUT] = jnp.dot(xt * wflat, smat[...],
                            preferred_element_type=_f32)


def _msgs(ea, xj, W1, b1, W2, b2, rmat, smat, nb, blk0, ne):
    full = lambda shape: pl.BlockSpec(shape, lambda i: (0, 0))
    return pl.pallas_call(
        _msgs_body,
        grid=(nb,),
        in_specs=[
            pl.BlockSpec((BE, D_EDGE), lambda i: (i + blk0, 0)),
            pl.BlockSpec((BE, 128), lambda i: (i, 0)),
            full((D_EDGE, HID)),
            full((1, HID)),
            full((HID, IN * OUT)),
            full((1, IN * OUT)),
            full((IN, IN * OUT)),
            full((IN * OUT, OUT)),
        ],
        out_specs=pl.BlockSpec((BE, 128), lambda i: (i, 0)),
        out_shape=jax.ShapeDtypeStruct((ne * LANE, 128), _f32),
        compiler_params=pltpu.CompilerParams(
            dimension_semantics=("parallel",)
        ),
    )(ea, xj, W1, b1, W2, b2, rmat, smat)


def _counts_body(dstidx_hbm, cnts_hbm, idx_v, onesb, zbuf, cacc, csem):
    c = lax.axis_index("c")
    s = lax.axis_index("s")
    w = s * NC + c

    def fill(i, carry):
        zbuf[i] = jnp.zeros((OUT,), _f32)
        return carry

    lax.fori_loop(0, STRIPE, fill, 0)

    def fill1(i, carry):
        onesb[i] = jnp.ones((OUT,), _f32)
        return carry

    lax.fori_loop(0, LANE, fill1, 0)
    pltpu.sync_copy(zbuf, cacc.at[pl.ds(s * STRIPE, STRIPE)])
    pltpu.sync_copy(dstidx_hbm.at[w], idx_v)
    plsc.subcore_barrier()

    def chunk(k, carry):
        for r in range(S_CH):
            pltpu.async_copy(
                onesb, cacc.at[idx_v.at[k * S_CH + r]], csem, add=True)
        for r in range(S_CH):
            pltpu.make_async_copy(
                cnts_hbm.at[0, pl.ds(0, LANE), pl.ds(0, OUT)], onesb,
                csem).wait()
        return carry

    lax.fori_loop(0, RPW // S_CH, chunk, 0)
    plsc.subcore_barrier()
    pltpu.sync_copy(cacc.at[pl.ds(s * STRIPE, STRIPE)],
                    cnts_hbm.at[c, pl.ds(s * STRIPE, STRIPE), pl.ds(0, OUT)])


_counts = functools.partial(
    pl.kernel,
    out_type=jax.ShapeDtypeStruct((NC, N_ACC, 128), _f32),
    mesh=plsc.VectorSubcoreMesh(core_axis_name="c", subcore_axis_name="s"),
    scratch_types=[
        pltpu.VMEM((RPW, LANE), jnp.int32),
        pltpu.VMEM((LANE, OUT), _f32),
        pltpu.VMEM((STRIPE, OUT), _f32),
        pltpu.VMEM_SHARED((N_ACC, OUT), _f32),
        pltpu.SemaphoreType.DMA,
    ],
    compiler_params=pltpu.CompilerParams(use_tc_tiling_on_sc=False),
)(_counts_body)


def _make_scatter(rpw, sch):
  def _scatter_body(msgs_hbm, dstidx_hbm, sums_hbm,
                    idx_v, mbuf0, mbuf1, zbuf, acc, ssem0, ssem1):
    c = lax.axis_index("c")
    s = lax.axis_index("s")
    w = s * NC + c

    def fill(i, carry):
        zbuf[i] = jnp.zeros((OUT,), _f32)
        return carry

    lax.fori_loop(0, STRIPE, fill, 0)
    pltpu.sync_copy(zbuf, acc.at[pl.ds(s * STRIPE, STRIPE)])
    pltpu.sync_copy(dstidx_hbm.at[w], idx_v)
    plsc.subcore_barrier()
    mbufs = (mbuf0, mbuf1)
    ssems = (ssem0, ssem1)

    def outer(k2, carry):
        for b in range(2):
            kk = k2 * 2 + b
            mb = mbufs[b]
            ss = ssems[b]

            @pl.when(kk >= 2)
            def _drain():
                pltpu.make_async_copy(
                    msgs_hbm.at[pl.ds(0, sch * LANE), pl.ds(0, OUT)],
                    mb, ss).wait()

            pltpu.sync_copy(
                msgs_hbm.at[pl.ds((w * rpw + kk * sch) * LANE,
                                  sch * LANE), pl.ds(0, OUT)], mb)
            for r in range(sch):
                pltpu.async_copy(
                    mb.at[pl.ds(r * LANE, LANE)],
                    acc.at[idx_v.at[kk * sch + r]],
                    ss,
                    add=True,
                )
        return carry

    lax.fori_loop(0, rpw // sch // 2, outer, 0)
    for b in range(2):
        pltpu.make_async_copy(
            msgs_hbm.at[pl.ds(0, sch * LANE), pl.ds(0, OUT)],
            mbufs[b], ssems[b]).wait()
    plsc.subcore_barrier()
    pltpu.sync_copy(acc.at[pl.ds(s * STRIPE, STRIPE)],
                    sums_hbm.at[c, pl.ds(s * STRIPE, STRIPE), pl.ds(0, OUT)])

  return functools.partial(
      pl.kernel,
      out_type=jax.ShapeDtypeStruct((NC, N_ACC, 128), _f32),
      mesh=plsc.VectorSubcoreMesh(core_axis_name="c", subcore_axis_name="s"),
      scratch_types=[
          pltpu.VMEM((rpw, LANE), jnp.int32),
          pltpu.VMEM((sch * LANE, OUT), _f32),
          pltpu.VMEM((sch * LANE, OUT), _f32),
          pltpu.VMEM((STRIPE, OUT), _f32),
          pltpu.VMEM_SHARED((N_ACC, OUT), _f32),
          pltpu.SemaphoreType.DMA,
          pltpu.SemaphoreType.DMA,
      ],
      compiler_params=pltpu.CompilerParams(use_tc_tiling_on_sc=False),
  )(_scatter_body)


def _final_body(s1_ref, s2_ref, cnts_ref, x_ref, root_ref, bias_ref,
                gamma_ref, beta_ref, out_ref):
    summ = (s1_ref[0] + s1_ref[1] + s2_ref[0] + s2_ref[1])[0:N, 0:OUT]
    cnt = (cnts_ref[0] + cnts_ref[1])[0:N, 0:OUT]
    aggr = summ / jnp.maximum(cnt, 1.0)
    xv = x_ref[...]
    h = aggr + jnp.dot(xv, root_ref[...], preferred_element_type=_f32) \
        + bias_ref[...]
    mu = jnp.mean(h, axis=0, keepdims=True)
    var = jnp.mean((h - mu) ** 2, axis=0, keepdims=True)
    hn = (h - mu) / jnp.sqrt(var + 1e-5) * gamma_ref[...] + beta_ref[...]
    out_ref[...] = xv + jnp.maximum(hn, 0.0)


def _final(s1, s2, cnts, x, root, bias, gamma, beta):
    return pl.pallas_call(
        _final_body,
        out_shape=jax.ShapeDtypeStruct((N, OUT), _f32),
    )(s1, s2, cnts, x, root, bias, gamma, beta)


def kernel(x, edge_index, edge_attr, W1, b1, W2, b2, root, bias, gamma, beta):
    src = edge_index[0].astype(jnp.int32)
    dst = edge_index[1].astype(jnp.int32)
    src_p = jnp.concatenate(
        [src, jnp.zeros((GIDX * LANE - E,), jnp.int32)]).reshape(GIDX, LANE)
    dst_p = jnp.concatenate(
        [dst, jnp.full((PAD,), N, jnp.int32)]).reshape(NW, RPW, LANE)

    cols = jnp.arange(IN * OUT)
    rmat = (cols[None, :] // OUT == jnp.arange(IN)[:, None]).astype(_f32)
    smat = (cols[:, None] % OUT == jnp.arange(OUT)[None, :]).astype(_f32)

    cnts = _counts(dst_p)

    HR = NW * RPW // 2
    NB1 = HR * LANE // BE
    NB2 = (E - HR * LANE + BE - 1) // BE
    dst_flat = dst_p.reshape(NW * RPW, LANE)
    dst_p1 = dst_flat[0:HR].reshape(NW, RPW // 2, LANE)
    dst_p2 = dst_flat[HR:2 * HR].reshape(NW, RPW // 2, LANE)

    gather1 = _make_gather(48, 32, HR, 0)
    gather2 = _make_gather(48, 32, HR, HR)
    scatter_h = _make_scatter(RPW // 2, 4)
    b1r = b1.reshape(1, HID)
    b2r = b2.reshape(1, IN * OUT)

    xj1 = gather1(x, src_p)
    xj2 = gather2(x, src_p)
    m1 = _msgs(edge_attr, xj1, W1, b1r, W2, b2r, rmat, smat, NB1, 0, HR)
    s1 = scatter_h(m1, dst_p1)
    m2 = _msgs(edge_attr, xj2, W1, b1r, W2, b2r, rmat, smat, NB2, NB1, HR)
    s2 = scatter_h(m2, dst_p2)

    return _final(s1, s2, cnts, x, root,
                  bias.reshape(1, OUT), gamma.reshape(1, OUT),
                  beta.reshape(1, OUT))


# speedup vs baseline: 1.2798x; 1.2798x over previous
import functools

import jax
import jax.numpy as jnp
from jax import lax
from jax.experimental import pallas as pl
from jax.experimental.pallas import tpu as pltpu
from jax.experimental.pallas import tpu_sc as plsc

N = 10000
E = 320000
IN = 16
OUT = 16
D_EDGE = 16
HID = 64

NC = 2
NS = 16
NW = NC * NS
LANE = 128
RPW = 80
E_PAD = NW * RPW * LANE
EW = E_PAD // 8
PAD = E_PAD - E

G_CH = 4
GA = 112
GB = 48
GIDX = NS * GA + NS * GB + (GA - GB)
S_CH = 8

N_ACC = 10240
NAW = N_ACC * 16 // 128
STRIPE = N_ACC // NS
SW = STRIPE * 16 // 128

BE = 4096
BEW = BE // 8
NB = (E + BE - 1) // BE

_f32 = jnp.float32


def _make_gather(ga, gb, ne, row0):
  def _gather_body(x_hbm, srcidx_hbm, xj_hbm, idx_v, gbuf0, gbuf1, gsem,
                   osem0, osem1):
    c = lax.axis_index("c")
    s = lax.axis_index("s")
    local_row = jnp.where(c == 0, s * ga, NS * ga + s * gb)
    n_outer = jnp.where(c == 0, ga // (2 * G_CH), gb // (2 * G_CH))
    pltpu.sync_copy(srcidx_hbm.at[pl.ds(row0 + local_row, ga)], idx_v)
    gbufs = (gbuf0, gbuf1)
    osems = (osem0, osem1)

    def outer(k2, carry):
        for b in range(2):
            kk = k2 * 2 + b
            gb = gbufs[b]
            os_ = osems[b]

            @pl.when(kk >= 2)
            def _drain():
                pltpu.make_async_copy(
                    gb, xj_hbm.at[pl.ds(0, G_CH * LANE), pl.ds(0, IN)],
                    os_).wait()

            descs = []
            for r in range(G_CH):
                descs.append(
                    pltpu.async_copy(
                        x_hbm.at[idx_v.at[kk * G_CH + r]],
                        gb.at[pl.ds(r * LANE, LANE)],
                        gsem,
                    )
                )
            for d in descs:
                d.wait()
            pltpu.async_copy(
                gb,
                xj_hbm.at[pl.ds((local_row + kk * G_CH) * LANE, G_CH * LANE),
                          pl.ds(0, IN)],
                os_,
            )
        return carry

    lax.fori_loop(0, n_outer, outer, 0)
    for b in range(2):
        pltpu.make_async_copy(
            gbufs[b], xj_hbm.at[pl.ds(0, G_CH * LANE), pl.ds(0, IN)],
            osems[b]).wait()

  return functools.partial(
      pl.kernel,
      out_type=jax.ShapeDtypeStruct((ne * LANE, 128), _f32),
      mesh=plsc.VectorSubcoreMesh(core_axis_name="c", subcore_axis_name="s"),
      scratch_types=[
          pltpu.VMEM((ga, LANE), jnp.int32),
          pltpu.VMEM((G_CH * LANE, IN), _f32),
          pltpu.VMEM((G_CH * LANE, IN), _f32),
          pltpu.SemaphoreType.DMA,
          pltpu.SemaphoreType.DMA,
          pltpu.SemaphoreType.DMA,
      ],
      compiler_params=pltpu.CompilerParams(use_tc_tiling_on_sc=False),
  )(_gather_body)


def _msgs_body(ea_t, xj, w1, b1, w2, b2, rmat, smat, out):
    h = jnp.maximum(
        lax.dot_general(ea_t[...], w1[...], (((0,), (0,)), ((), ())),
                        preferred_element_type=_f32) + b1[...], 0.0
    )
    wflat = jnp.dot(h, w2[...], preferred_element_type=_f32) + b2[...]
    xt = jnp.dot(xj[:, 0:IN], rmat[...], preferred_element_type=_f32)
    out[:, 0:OUT] = jnp.dot(xt * wflat, smat[...],
                            preferred_element_type=_f32)


def _msgs(ea, xj, W1, b1, W2, b2, rmat, smat, nb, blk0, ne):
    full = lambda shape: pl.BlockSpec(shape, lambda i: (0, 0))
    return pl.pallas_call(
        _msgs_body,
        grid=(nb,),
        in_specs=[
            pl.BlockSpec((D_EDGE, BE), lambda i: (0, i + blk0)),
            pl.BlockSpec((BE, 128), lambda i: (i, 0)),
            full((D_EDGE, HID)),
            full((1, HID)),
            full((HID, IN * OUT)),
            full((1, IN * OUT)),
            full((IN, IN * OUT)),
            full((IN * OUT, OUT)),
        ],
        out_specs=pl.BlockSpec((BE, 128), lambda i: (i, 0)),
        out_shape=jax.ShapeDtypeStruct((ne * LANE, 128), _f32),
        compiler_params=pltpu.CompilerParams(
            dimension_semantics=("parallel",),
            fuse_transposed_lhs_in_matmul=True,
        ),
    )(ea, xj, W1, b1, W2, b2, rmat, smat)


def _counts_body(dstidx_hbm, cnts_hbm, idx_v, onesb, zbuf, cacc, csem):
    c = lax.axis_index("c")
    s = lax.axis_index("s")
    w = s * NC + c

    def fill(i, carry):
        zbuf[i] = jnp.zeros((OUT,), _f32)
        return carry

    lax.fori_loop(0, STRIPE, fill, 0)

    def fill1(i, carry):
        onesb[i] = jnp.ones((OUT,), _f32)
        return carry

    lax.fori_loop(0, LANE, fill1, 0)
    pltpu.sync_copy(zbuf, cacc.at[pl.ds(s * STRIPE, STRIPE)])
    pltpu.sync_copy(dstidx_hbm.at[w], idx_v)
    plsc.subcore_barrier()

    def chunk(k, carry):
        for r in range(S_CH):
            pltpu.async_copy(
                onesb, cacc.at[idx_v.at[k * S_CH + r]], csem, add=True)
        for r in range(S_CH):
            pltpu.make_async_copy(
                cnts_hbm.at[0, pl.ds(0, LANE), pl.ds(0, OUT)], onesb,
                csem).wait()
        return carry

    lax.fori_loop(0, RPW // S_CH, chunk, 0)
    plsc.subcore_barrier()
    pltpu.sync_copy(cacc.at[pl.ds(s * STRIPE, STRIPE)],
                    cnts_hbm.at[c, pl.ds(s * STRIPE, STRIPE), pl.ds(0, OUT)])


_counts = functools.partial(
    pl.kernel,
    out_type=jax.ShapeDtypeStruct((NC, N_ACC, 128), _f32),
    mesh=plsc.VectorSubcoreMesh(core_axis_name="c", subcore_axis_name="s"),
    scratch_types=[
        pltpu.VMEM((RPW, LANE), jnp.int32),
        pltpu.VMEM((LANE, OUT), _f32),
        pltpu.VMEM((STRIPE, OUT), _f32),
        pltpu.VMEM_SHARED((N_ACC, OUT), _f32),
        pltpu.SemaphoreType.DMA,
    ],
    compiler_params=pltpu.CompilerParams(use_tc_tiling_on_sc=False),
)(_counts_body)


def _make_scatter(rpw, sch):
  def _scatter_body(msgs_hbm, dstidx_hbm, sums_hbm,
                    idx_v, mbuf0, mbuf1, zbuf, acc, ssem0, ssem1):
    c = lax.axis_index("c")
    s = lax.axis_index("s")
    w = s * NC + c

    def fill(i, carry):
        zbuf[i] = jnp.zeros((OUT,), _f32)
        return carry

    lax.fori_loop(0, STRIPE, fill, 0)
    pltpu.sync_copy(zbuf, acc.at[pl.ds(s * STRIPE, STRIPE)])
    pltpu.sync_copy(dstidx_hbm.at[w], idx_v)
    plsc.subcore_barrier()
    mbufs = (mbuf0, mbuf1)
    ssems = (ssem0, ssem1)

    def outer(k2, carry):
        for b in range(2):
            kk = k2 * 2 + b
            mb = mbufs[b]
            ss = ssems[b]

            @pl.when(kk >= 2)
            def _drain():
                pltpu.make_async_copy(
                    msgs_hbm.at[pl.ds(0, sch * LANE), pl.ds(0, OUT)],
                    mb, ss).wait()

            pltpu.sync_copy(
                msgs_hbm.at[pl.ds((w * rpw + kk * sch) * LANE,
                                  sch * LANE), pl.ds(0, OUT)], mb)
            for r in range(sch):
                pltpu.async_copy(
                    mb.at[pl.ds(r * LANE, LANE)],
                    acc.at[idx_v.at[kk * sch + r]],
                    ss,
                    add=True,
                )
        return carry

    lax.fori_loop(0, rpw // sch // 2, outer, 0)
    for b in range(2):
        pltpu.make_async_copy(
            msgs_hbm.at[pl.ds(0, sch * LANE), pl.ds(0, OUT)],
            mbufs[b], ssems[b]).wait()
    plsc.subcore_barrier()
    pltpu.sync_copy(acc.at[pl.ds(s * STRIPE, STRIPE)],
                    sums_hbm.at[c, pl.ds(s * STRIPE, STRIPE), pl.ds(0, OUT)])

  return functools.partial(
      pl.kernel,
      out_type=jax.ShapeDtypeStruct((NC, N_ACC, 128), _f32),
      mesh=plsc.VectorSubcoreMesh(core_axis_name="c", subcore_axis_name="s"),
      scratch_types=[
          pltpu.VMEM((rpw, LANE), jnp.int32),
          pltpu.VMEM((sch * LANE, OUT), _f32),
          pltpu.VMEM((sch * LANE, OUT), _f32),
          pltpu.VMEM((STRIPE, OUT), _f32),
          pltpu.VMEM_SHARED((N_ACC, OUT), _f32),
          pltpu.SemaphoreType.DMA,
          pltpu.SemaphoreType.DMA,
      ],
      compiler_params=pltpu.CompilerParams(use_tc_tiling_on_sc=False),
  )(_scatter_body)


def _final_body(s1_ref, s2_ref, cnts_ref, x_ref, root_ref, bias_ref,
                gamma_ref, beta_ref, out_ref):
    summ = (s1_ref[0] + s1_ref[1] + s2_ref[0] + s2_ref[1])[0:N, 0:OUT]
    cnt = (cnts_ref[0] + cnts_ref[1])[0:N, 0:OUT]
    aggr = summ / jnp.maximum(cnt, 1.0)
    xv = x_ref[...]
    h = aggr + jnp.dot(xv, root_ref[...], preferred_element_type=_f32) \
        + bias_ref[...]
    mu = jnp.mean(h, axis=0, keepdims=True)
    var = jnp.mean((h - mu) ** 2, axis=0, keepdims=True)
    hn = (h - mu) / jnp.sqrt(var + 1e-5) * gamma_ref[...] + beta_ref[...]
    out_ref[...] = xv + jnp.maximum(hn, 0.0)


def _final(s1, s2, cnts, x, root, bias, gamma, beta):
    return pl.pallas_call(
        _final_body,
        out_shape=jax.ShapeDtypeStruct((N, OUT), _f32),
    )(s1, s2, cnts, x, root, bias, gamma, beta)


def kernel(x, edge_index, edge_attr, W1, b1, W2, b2, root, bias, gamma, beta):
    src = edge_index[0].astype(jnp.int32)
    dst = edge_index[1].astype(jnp.int32)
    src_p = jnp.concatenate(
        [src, jnp.zeros((GIDX * LANE - E,), jnp.int32)]).reshape(GIDX, LANE)
    dst_p = jnp.concatenate(
        [dst, jnp.full((PAD,), N, jnp.int32)]).reshape(NW, RPW, LANE)

    cols = jnp.arange(IN * OUT)
    rmat = (cols[None, :] // OUT == jnp.arange(IN)[:, None]).astype(_f32)
    smat = (cols[:, None] % OUT == jnp.arange(OUT)[None, :]).astype(_f32)

    cnts = _counts(dst_p)

    HR = NW * RPW // 2
    NB1 = HR * LANE // BE
    NB2 = (E - HR * LANE + BE - 1) // BE
    dst_flat = dst_p.reshape(NW * RPW, LANE)
    dst_p1 = dst_flat[0:HR].reshape(NW, RPW // 2, LANE)
    dst_p2 = dst_flat[HR:2 * HR].reshape(NW, RPW // 2, LANE)

    gather1 = _make_gather(48, 32, HR, 0)
    gather2 = _make_gather(48, 32, HR, HR)
    scatter_h = _make_scatter(RPW // 2, 4)
    b1r = b1.reshape(1, HID)
    b2r = b2.reshape(1, IN * OUT)

    ea_t = jnp.transpose(edge_attr)
    xj1 = gather1(x, src_p)
    xj2 = gather2(x, src_p)
    m1 = _msgs(ea_t, xj1, W1, b1r, W2, b2r, rmat, smat, NB1, 0, HR)
    s1 = scatter_h(m1, dst_p1)
    m2 = _msgs(ea_t, xj2, W1, b1r, W2, b2r, rmat, smat, NB2, NB1, HR)
    s2 = scatter_h(m2, dst_p2)

    return _final(s1, s2, cnts, x, root,
                  bias.reshape(1, OUT), gamma.reshape(1, OUT),
                  beta.reshape(1, OUT))
